# Initial kernel scaffold; baseline (speedup 1.0000x reference)
import jax
import jax.numpy as jnp
from jax.experimental import pallas as pl


def kernel(inputs, edge_index, lap_vals, W, b):
    raise NotImplementedError("write your pallas kernel here")


# speedup vs baseline: 2.2605x; 2.2605x over previous
import functools

import jax
import jax.numpy as jnp
from jax import lax
from jax.experimental import pallas as pl
from jax.experimental.pallas import tpu as pltpu
from jax.experimental.pallas import tpu_sc as plsc

_B, _M, _FIN, _FOUT, _K = 4, 10000, 128, 128, 4
_E = 320000

_NC, _NS, _L = 2, 16, 16
_NB = 80
_TEB = _E // _NS
_NBATCH = _TEB // _NB
_RBLK = 80
_NRB = _M // _RBLK
_RITER = (_NRB + _NS - 1) // _NS
_CPB = _B // _NC


def _sc_body(x0, row, col, lap, x1, x2, x3,
             acc, colv, cofv, rowv, lapv, rowsv, zerov, blkv, prevv, sem):
    c = lax.axis_index("c")
    s = lax.axis_index("s")
    zero16 = jnp.zeros((_L,), jnp.float32)

    def _zb(r, carry):
        for j in range(_FIN // _L):
            zerov[r, pl.ds(j * _L, _L)] = zero16
        return carry
    lax.fori_loop(0, _RBLK, _zb, 0)
    for r in range(_RITER):
        blk = s + r * _NS
        @pl.when(blk < _NRB)
        def _():
            pltpu.sync_copy(zerov, acc.at[pl.ds(blk * _RBLK, _RBLK)])
    plsc.subcore_barrier()

    def _spmm(src, dst, prev, b, factor):
        boff = b * _M

        def _batch(i, carry):
            base = s * _TEB + i * _NB
            pltpu.sync_copy(row.at[pl.ds(base, _NB)], rowv)
            pltpu.sync_copy(col.at[pl.ds(base, _NB)], colv)
            pltpu.sync_copy(lap.at[pl.ds(base, _NB)], lapv)
            for j in range(_NB // _L):
                sl = pl.ds(j * _L, _L)
                cofv[sl] = colv[sl] + boff
                if factor != 1.0:
                    lapv[sl] = lapv[sl] * factor
            pltpu.async_copy(src.at[cofv], rowsv, sem).wait()

            def _edge_group(g, ecarry):
                lap16 = lapv[pl.ds(g * _L, _L)]
                for t in range(_L):
                    lv = lap16[t]
                    e = g * _L + t
                    for j in range(_FIN // _L):
                        sl = pl.ds(j * _L, _L)
                        rowsv[e, sl] = rowsv[e, sl] * lv
                return ecarry
            lax.fori_loop(0, _NB // _L, _edge_group, 0)
            pltpu.sync_copy(rowsv, acc.at[rowv], add=True)
            return carry
        lax.fori_loop(0, _NBATCH, _batch, 0)
        plsc.subcore_barrier()

        for r in range(_RITER):
            blk = s + r * _NS

            @pl.when(blk < _NRB)
            def _():
                r0 = blk * _RBLK
                pltpu.sync_copy(acc.at[pl.ds(r0, _RBLK)], blkv)
                pltpu.sync_copy(zerov, acc.at[pl.ds(r0, _RBLK)])
                if prev is not None:
                    pltpu.sync_copy(prev.at[pl.ds(boff + r0, _RBLK)], prevv)

                    def _sub(rr, scarry):
                        for j in range(_FIN // _L):
                            sl = pl.ds(j * _L, _L)
                            blkv[rr, sl] = blkv[rr, sl] - prevv[rr, sl]
                        return scarry
                    lax.fori_loop(0, _RBLK, _sub, 0)
                pltpu.sync_copy(blkv, dst.at[pl.ds(boff + r0, _RBLK)])
        plsc.subcore_barrier()

    for bi in range(_CPB):
        b = c * _CPB + bi
        _spmm(x0, x1, None, b, 1.0)
        _spmm(x1, x2, x0, b, 2.0)
        _spmm(x2, x3, x1, b, 2.0)


_spmm3 = pl.kernel(
    _sc_body,
    out_type=[jax.ShapeDtypeStruct((_B * _M, _FIN), jnp.float32)] * 3,
    mesh=plsc.VectorSubcoreMesh(core_axis_name="c", subcore_axis_name="s",
                                num_cores=_NC, num_subcores=_NS),
    scratch_types=[
        pltpu.VMEM_SHARED((_M, _FIN), jnp.float32),
        pltpu.VMEM((_NB,), jnp.int32),
        pltpu.VMEM((_NB,), jnp.int32),
        pltpu.VMEM((_NB,), jnp.int32),
        pltpu.VMEM((_NB,), jnp.float32),
        pltpu.VMEM((_NB, _FIN), jnp.float32),
        pltpu.VMEM((_RBLK, _FIN), jnp.float32),
        pltpu.VMEM((_RBLK, _FIN), jnp.float32),
        pltpu.VMEM((_RBLK, _FIN), jnp.float32),
        pltpu.SemaphoreType.DMA,
    ],
)


_BMB = 2000
_NBM = _B * _M // _BMB


def _tc_body(x0, x1, x2, x3, w, bias, out):
    acc = jnp.dot(x0[...], w[:, 0, :], preferred_element_type=jnp.float32)
    acc += jnp.dot(x1[...], w[:, 1, :], preferred_element_type=jnp.float32)
    acc += jnp.dot(x2[...], w[:, 2, :], preferred_element_type=jnp.float32)
    acc += jnp.dot(x3[...], w[:, 3, :], preferred_element_type=jnp.float32)
    out[...] = acc + bias[0, 0, :]


_cheb_out = pl.pallas_call(
    _tc_body,
    grid=(_NBM,),
    in_specs=[
        pl.BlockSpec((_BMB, _FIN), lambda i: (i, 0)),
        pl.BlockSpec((_BMB, _FIN), lambda i: (i, 0)),
        pl.BlockSpec((_BMB, _FIN), lambda i: (i, 0)),
        pl.BlockSpec((_BMB, _FIN), lambda i: (i, 0)),
        pl.BlockSpec((_FIN, _K, _FOUT), lambda i: (0, 0, 0)),
        pl.BlockSpec((1, 1, _FOUT), lambda i: (0, 0, 0)),
    ],
    out_specs=pl.BlockSpec((_BMB, _FOUT), lambda i: (i, 0)),
    out_shape=jax.ShapeDtypeStruct((_B * _M, _FOUT), jnp.float32),
)


def kernel(inputs, edge_index, lap_vals, W, b):
    x0 = inputs.reshape(_B * _M, _FIN)
    row = edge_index[0]
    col = edge_index[1]
    x1, x2, x3 = _spmm3(x0, row, col, lap_vals)
    out = _cheb_out(x0, x1, x2, x3, W, b)
    return out.reshape(_B, _M, _FOUT)


# speedup vs baseline: 6.0647x; 2.6828x over previous
import functools

import jax
import jax.numpy as jnp
from jax import lax
from jax.experimental import pallas as pl
from jax.experimental.pallas import tpu as pltpu
from jax.experimental.pallas import tpu_sc as plsc

_B, _M, _FIN, _FOUT, _K = 4, 10000, 128, 128, 4
_E = 320000

_NC, _NS, _L = 2, 16, 16
_NB = 80
_TEB = _E // _NS
_NBATCH = _TEB // _NB
_NPAIR = _NBATCH // 2
_RBLK = 80
_NRB = _M // _RBLK
_RITER = (_NRB + _NS - 1) // _NS
_CPB = _B // _NC


def _sc_body(x0, rc3, lap3, g1, g2, g3,
             acc, idx_a, idx_b, lapr_a, lapr_b, rowv_a, cofv_a, lapv_a,
             rowv_b, cofv_b, lapv_b, rows_a, rows_b, zerov,
             isa, isb, gsa, gsb, ssa, ssb):
    c = lax.axis_index("c")
    s = lax.axis_index("s")
    zero16 = jnp.zeros((_L,), jnp.float32)

    def _zb(r, carry):
        for j in range(_FIN // _L):
            zerov[r, pl.ds(j * _L, _L)] = zero16
        return carry
    lax.fori_loop(0, _RBLK, _zb, 0)
    for r in range(_RITER):
        blk = s + r * _NS

        @pl.when(blk < _NRB)
        def _():
            pltpu.sync_copy(zerov, acc.at[pl.ds(blk * _RBLK, _RBLK)])
    plsc.subcore_barrier()

    def _unpack(rc, lapr, rowv, cofv, lapv, boff):
        for j in range(_NB // _L):
            sl = pl.ds(j * _L, _L)
            v = rc[sl]
            rowv[sl] = lax.shift_right_logical(v, 16)
            cofv[sl] = jnp.bitwise_and(v, 0xFFFF) + boff
            lapv[sl] = lapr[sl]

    def _scale(buf, lapv):
        def _grp(g, carry):
            lap16 = lapv[pl.ds(g * _L, _L)]
            for t in range(_L):
                lv = lap16[t]
                e = g * _L + t
                for j in range(_FIN // _L):
                    sl = pl.ds(j * _L, _L)
                    buf[e, sl] = buf[e, sl] * lv
            return carry
        lax.fori_loop(0, _NB // _L, _grp, 0)

    def _pass(src, dst, boff):
        pltpu.async_copy(rc3.at[s, 0], idx_a, isa)
        pltpu.async_copy(lap3.at[s, 0], lapr_a, isa)
        pltpu.async_copy(rc3.at[s, 1], idx_b, isb)
        pltpu.async_copy(lap3.at[s, 1], lapr_b, isb)
        pltpu.make_async_copy(rc3.at[s, 0], idx_a, isa).wait()
        pltpu.make_async_copy(lap3.at[s, 0], lapr_a, isa).wait()
        _unpack(idx_a, lapr_a, rowv_a, cofv_a, lapv_a, boff)
        pltpu.async_copy(src.at[cofv_a], rows_a, gsa)
        pltpu.make_async_copy(rc3.at[s, 1], idx_b, isb).wait()
        pltpu.make_async_copy(lap3.at[s, 1], lapr_b, isb).wait()
        _unpack(idx_b, lapr_b, rowv_b, cofv_b, lapv_b, boff)
        pltpu.async_copy(src.at[cofv_b], rows_b, gsb)
        pltpu.async_copy(rc3.at[s, 2], idx_a, isa)
        pltpu.async_copy(lap3.at[s, 2], lapr_a, isa)
        pltpu.async_copy(rc3.at[s, 3], idx_b, isb)
        pltpu.async_copy(lap3.at[s, 3], lapr_b, isb)

        def _pair(ii, carry):
            i0 = 2 * ii
            i1 = i0 + 1
            pltpu.make_async_copy(src.at[pl.ds(0, _NB)], rows_a, gsa).wait()
            _scale(rows_a, lapv_a)
            pltpu.async_copy(rows_a, acc.at[rowv_a], ssa, add=True)
            pltpu.make_async_copy(src.at[pl.ds(0, _NB)], rows_b, gsb).wait()
            _scale(rows_b, lapv_b)
            pltpu.async_copy(rows_b, acc.at[rowv_b], ssb, add=True)

            @pl.when(ii < _NPAIR - 1)
            def _():
                pltpu.make_async_copy(rc3.at[s, 0], idx_a, isa).wait()
                pltpu.make_async_copy(lap3.at[s, 0], lapr_a, isa).wait()
                pltpu.make_async_copy(rows_a, acc.at[pl.ds(0, _NB)], ssa).wait()
                _unpack(idx_a, lapr_a, rowv_a, cofv_a, lapv_a, boff)
                pltpu.async_copy(src.at[cofv_a], rows_a, gsa)
                pltpu.make_async_copy(rc3.at[s, 1], idx_b, isb).wait()
                pltpu.make_async_copy(lap3.at[s, 1], lapr_b, isb).wait()
                pltpu.make_async_copy(rows_b, acc.at[pl.ds(0, _NB)], ssb).wait()
                _unpack(idx_b, lapr_b, rowv_b, cofv_b, lapv_b, boff)
                pltpu.async_copy(src.at[cofv_b], rows_b, gsb)

                @pl.when(ii < _NPAIR - 2)
                def _():
                    pltpu.async_copy(rc3.at[s, i0 + 4], idx_a, isa)
                    pltpu.async_copy(lap3.at[s, i0 + 4], lapr_a, isa)
                    pltpu.async_copy(rc3.at[s, i1 + 4], idx_b, isb)
                    pltpu.async_copy(lap3.at[s, i1 + 4], lapr_b, isb)
            return carry
        lax.fori_loop(0, _NPAIR, _pair, 0)
        pltpu.make_async_copy(rows_a, acc.at[pl.ds(0, _NB)], ssa).wait()
        pltpu.make_async_copy(rows_b, acc.at[pl.ds(0, _NB)], ssb).wait()
        plsc.subcore_barrier()

        for r in range(_RITER):
            blk = s + r * _NS

            @pl.when(blk < _NRB)
            def _():
                r0 = blk * _RBLK
                pltpu.sync_copy(acc.at[pl.ds(r0, _RBLK)],
                                dst.at[pl.ds(boff + r0, _RBLK)])
                pltpu.sync_copy(zerov, acc.at[pl.ds(r0, _RBLK)])
        plsc.subcore_barrier()

    def _chunk(bi, carry):
        boff = (c * _CPB + bi) * _M
        _pass(x0, g1, boff)
        _pass(g1, g2, boff)
        _pass(g2, g3, boff)
        return carry
    lax.fori_loop(0, _CPB, _chunk, 0)


_spmm3 = pl.kernel(
    _sc_body,
    out_type=[jax.ShapeDtypeStruct((_B * _M, _FIN), jnp.float32)] * 3,
    mesh=plsc.VectorSubcoreMesh(core_axis_name="c", subcore_axis_name="s",
                                num_cores=_NC, num_subcores=_NS),
    scratch_types=[
        pltpu.VMEM_SHARED((_M, _FIN), jnp.float32),
        pltpu.VMEM((_NB,), jnp.int32),
        pltpu.VMEM((_NB,), jnp.int32),
        pltpu.VMEM((_NB,), jnp.float32),
        pltpu.VMEM((_NB,), jnp.float32),
        pltpu.VMEM((_NB,), jnp.int32),
        pltpu.VMEM((_NB,), jnp.int32),
        pltpu.VMEM((_NB,), jnp.float32),
        pltpu.VMEM((_NB,), jnp.int32),
        pltpu.VMEM((_NB,), jnp.int32),
        pltpu.VMEM((_NB,), jnp.float32),
        pltpu.VMEM((_NB, _FIN), jnp.float32),
        pltpu.VMEM((_NB, _FIN), jnp.float32),
        pltpu.VMEM((_RBLK, _FIN), jnp.float32),
        pltpu.SemaphoreType.DMA,
        pltpu.SemaphoreType.DMA,
        pltpu.SemaphoreType.DMA,
        pltpu.SemaphoreType.DMA,
        pltpu.SemaphoreType.DMA,
        pltpu.SemaphoreType.DMA,
    ],
)


_BMB = 2000
_NBM = _B * _M // _BMB


def _tc_body(x0, g1, g2, g3, w, bias, out):
    acc = jnp.dot(x0[...], w[:, 0, :], preferred_element_type=jnp.float32)
    acc += jnp.dot(g1[...], w[:, 1, :], preferred_element_type=jnp.float32)
    acc += jnp.dot(g2[...], w[:, 2, :], preferred_element_type=jnp.float32)
    acc += jnp.dot(g3[...], w[:, 3, :], preferred_element_type=jnp.float32)
    out[...] = acc + bias[0, 0, :]


_cheb_out = pl.pallas_call(
    _tc_body,
    grid=(_NBM,),
    in_specs=[
        pl.BlockSpec((_BMB, _FIN), lambda i: (i, 0)),
        pl.BlockSpec((_BMB, _FIN), lambda i: (i, 0)),
        pl.BlockSpec((_BMB, _FIN), lambda i: (i, 0)),
        pl.BlockSpec((_BMB, _FIN), lambda i: (i, 0)),
        pl.BlockSpec((_FIN, _K, _FOUT), lambda i: (0, 0, 0)),
        pl.BlockSpec((1, 1, _FOUT), lambda i: (0, 0, 0)),
    ],
    out_specs=pl.BlockSpec((_BMB, _FOUT), lambda i: (i, 0)),
    out_shape=jax.ShapeDtypeStruct((_B * _M, _FOUT), jnp.float32),
)


def kernel(inputs, edge_index, lap_vals, W, b):
    x0 = inputs.reshape(_B * _M, _FIN)
    rc3 = (jnp.left_shift(edge_index[0], 16)
           | edge_index[1]).reshape(_NS, _NBATCH, _NB)
    lap3 = lap_vals.reshape(_NS, _NBATCH, _NB)
    g1, g2, g3 = _spmm3(x0, rc3, lap3)
    wt = jnp.stack([W[:, 0, :] - W[:, 2, :],
                    W[:, 1, :] - 3.0 * W[:, 3, :],
                    2.0 * W[:, 2, :],
                    4.0 * W[:, 3, :]], axis=1)
    out = _cheb_out(x0, g1, g2, g3, wt, b)
    return out.reshape(_B, _M, _FOUT)
